# baseline (speedup 1.0000x reference)
import functools

import jax
import jax.numpy as jnp
from jax import lax
from jax.experimental import pallas as pl
from jax.experimental.pallas import tpu as pltpu
from jax.experimental.pallas import tpu_sc as plsc

_TV = 2048
_NEG = -1e30
_DOT_T = (((0,), (1,)), ((), ()))


def _make_sc_gather(n, d):
    info = plsc.get_sparse_core_info()
    nc, ns = info.num_cores, info.num_subcores
    nw = nc * ns
    assert n % nw == 0
    per_w = n // nw
    chunk = 128
    assert per_w % chunk == 0
    nchunk = per_w // chunk
    mesh = plsc.VectorSubcoreMesh(core_axis_name="c", subcore_axis_name="s")

    @functools.partial(
        pl.kernel,
        mesh=mesh,
        out_type=jax.ShapeDtypeStruct((n, d), jnp.float32),
        scratch_types=[
            pltpu.VMEM((per_w,), jnp.int32),
            pltpu.VMEM((per_w, d), jnp.float32),
            pltpu.SemaphoreType.DMA,
        ],
    )
    def gather(table_hbm, idx_hbm, out_hbm, idx_v, rows_v, sem):
        wid = lax.axis_index("s") * nc + lax.axis_index("c")
        base = wid * per_w
        pltpu.sync_copy(idx_hbm.at[pl.ds(base, per_w)], idx_v)
        copies = [
            pltpu.async_copy(
                table_hbm.at[idx_v.at[pl.ds(c * chunk, chunk)]],
                rows_v.at[pl.ds(c * chunk, chunk)],
                sem,
            )
            for c in range(nchunk)
        ]
        for cp in copies:
            cp.wait()
        pltpu.sync_copy(rows_v, out_hbm.at[pl.ds(base, per_w)])

    return gather


def _mc_body(w2_ref, mc_ref, *, tv, vocab):
    j = pl.program_id(0)
    w2 = w2_ref[...]
    colsq = jnp.sum(w2 * w2, axis=0, keepdims=True)
    col = lax.broadcasted_iota(jnp.int32, (1, tv), 1) + j * tv
    colsq = jnp.where(col < vocab, colsq, 0.0)
    mc = jnp.sqrt(jnp.max(colsq, axis=1, keepdims=True))
    mc_ref[...] = jnp.broadcast_to(mc, mc_ref.shape)


def _h_body(x_ref, w1_ref, b1_ref, h2_ref, hm_ref, hacc_ref, *, nsteps, cpg, k2):
    c = pl.program_id(0)

    @pl.when(c == 0)
    def _():
        hacc_ref[...] = jnp.zeros(hacc_ref.shape, jnp.float32)

    b = hacc_ref.shape[0]
    acc = hacc_ref[...]
    for i in range(cpg):
        acc += jnp.dot(x_ref[i * b:(i + 1) * b, :], w1_ref[i],
                       preferred_element_type=jnp.float32)
    hacc_ref[...] = acc

    @pl.when(c == nsteps - 1)
    def _():
        h = jnp.maximum(acc + b1_ref[...], 0.0)
        extra = lax.broadcasted_iota(jnp.int32, (b, k2 - h.shape[1]), 1)
        h2 = jnp.concatenate([h, jnp.where(extra == 0, 1.0, 0.0)], axis=1)
        h2_ref[...] = h2
        hsq = jnp.sum(h2 * h2, axis=1, keepdims=True)
        hm = jnp.sqrt(jnp.max(hsq, axis=0, keepdims=True))
        hm_ref[...] = jnp.broadcast_to(hm, hm_ref.shape)


def _pass1_body(h2_ref, w2_ref, mc_ref, hm_ref, lse_ref, u_ref, s_ref, *, nv):
    j = pl.program_id(0)
    lt = lax.dot_general(w2_ref[...], h2_ref[...], _DOT_T,
                         preferred_element_type=jnp.float32)
    ub = mc_ref[0, 0:1, 0:1] * hm_ref[0:1, 0:1]
    s_j = jnp.sum(jnp.exp(lt - ub), axis=0, keepdims=True)

    @pl.when(j == 0)
    def _():
        u_ref[...] = jnp.full(u_ref.shape, _NEG, jnp.float32)
        s_ref[...] = jnp.zeros(s_ref.shape, jnp.float32)

    u_old = u_ref[0:1, 0:1]
    u_new = jnp.maximum(u_old, ub)
    s_new = (s_ref[...] * jnp.exp(u_old - u_new)
             + s_j * jnp.exp(ub - u_new))
    u_ref[...] = jnp.broadcast_to(u_new, u_ref.shape)
    s_ref[...] = s_new

    @pl.when(j == nv - 1)
    def _():
        lse = u_new + jnp.log(s_new)
        lse_ref[...] = jnp.broadcast_to(lse, lse_ref.shape)


def _pass2_body(h2_ref, w2_ref, lse_ref, out_ref):
    lt = lax.dot_general(w2_ref[...], h2_ref[...], _DOT_T,
                         preferred_element_type=jnp.float32)
    out_ref[...] = lt - lse_ref[:1, :]


def _tc_fused(rows, W1p3, b1, W2a, mc, vocab):
    ctx = W1p3.shape[0]
    b = rows.shape[0] // ctx
    k2 = W2a.shape[0]
    hid = W1p3.shape[2]
    dp = W1p3.shape[1]
    tv = _TV
    nv = W2a.shape[1] // tv
    cpg = 4
    nsteps = ctx // cpg

    h2, _hm = pl.pallas_call(
        functools.partial(_h_body, nsteps=nsteps, cpg=cpg, k2=k2),
        grid=(nsteps,),
        in_specs=[
            pl.BlockSpec((cpg * b, dp), lambda c: (c, 0)),
            pl.BlockSpec((cpg, dp, hid), lambda c: (c, 0, 0)),
            pl.BlockSpec((1, hid), lambda c: (0, 0)),
        ],
        out_specs=[
            pl.BlockSpec((b, k2), lambda c: (0, 0)),
            pl.BlockSpec((1, 128), lambda c: (0, 0)),
        ],
        out_shape=[
            jax.ShapeDtypeStruct((b, k2), jnp.float32),
            jax.ShapeDtypeStruct((1, 128), jnp.float32),
        ],
        scratch_shapes=[pltpu.VMEM((b, hid), jnp.float32)],
        compiler_params=pltpu.CompilerParams(
            dimension_semantics=("arbitrary",),
        ),
    )(rows, W1p3, b1.reshape(1, -1))

    lse = pl.pallas_call(
        functools.partial(_pass1_body, nv=nv),
        grid=(nv,),
        in_specs=[
            pl.BlockSpec((b, k2), lambda j: (0, 0)),
            pl.BlockSpec((k2, tv), lambda j: (0, j)),
            pl.BlockSpec((1, 1, 128), lambda j: (j, 0, 0)),
            pl.BlockSpec((1, 128), lambda j: (0, 0)),
        ],
        out_specs=pl.BlockSpec((8, b), lambda j: (0, 0)),
        out_shape=jax.ShapeDtypeStruct((8, b), jnp.float32),
        scratch_shapes=[
            pltpu.VMEM((1, 128), jnp.float32),
            pltpu.VMEM((1, b), jnp.float32),
        ],
        compiler_params=pltpu.CompilerParams(
            dimension_semantics=("arbitrary",),
        ),
    )(h2, W2a, mc, _hm)

    out_t = pl.pallas_call(
        _pass2_body,
        grid=(nv,),
        in_specs=[
            pl.BlockSpec((b, k2), lambda j: (0, 0)),
            pl.BlockSpec((k2, tv), lambda j: (0, j)),
            pl.BlockSpec((8, b), lambda j: (0, 0)),
        ],
        out_specs=pl.BlockSpec((tv, b), lambda j: (j, 0)),
        out_shape=jax.ShapeDtypeStruct((vocab, b), jnp.float32),
        compiler_params=pltpu.CompilerParams(
            dimension_semantics=("arbitrary",),
        ),
    )(h2, W2a, lse)
    return out_t.T


def kernel(seq, emb, W1, b1, W2, b2):
    b, ctx = seq.shape
    d = emb.shape[1]
    hid = W1.shape[1]
    vocab = W2.shape[1]
    tv = _TV
    nv = pl.cdiv(vocab, tv)
    vpad = nv * tv
    k2 = hid + 8

    dp = 128
    emb_p = jnp.pad(emb, ((0, 0), (0, dp - d)))
    W1p3 = jnp.pad(W1.reshape(ctx, d, hid), ((0, 0), (0, dp - d), (0, 0)))

    bias_row = jnp.concatenate(
        [b2[None, :], jnp.full((1, vpad - vocab), _NEG, jnp.float32)], axis=1)
    W2a = jnp.concatenate(
        [jnp.pad(W2, ((0, 0), (0, vpad - vocab))),
         bias_row,
         jnp.zeros((k2 - hid - 1, vpad), jnp.float32)], axis=0)

    mc = pl.pallas_call(
        functools.partial(_mc_body, tv=tv, vocab=vocab),
        grid=(nv,),
        in_specs=[pl.BlockSpec((k2, tv), lambda j: (0, j))],
        out_specs=pl.BlockSpec((1, 1, 128), lambda j: (j, 0, 0)),
        out_shape=jax.ShapeDtypeStruct((nv, 1, 128), jnp.float32),
        compiler_params=pltpu.CompilerParams(
            dimension_semantics=("arbitrary",),
        ),
    )(W2a)

    seq_cm = seq.T.reshape(-1)
    gather = _make_sc_gather(b * ctx, dp)
    rows = gather(emb_p, seq_cm)
    return _tc_fused(rows, W1p3, b1, W2a, mc, vocab)


# speedup vs baseline: 1.0977x; 1.0977x over previous
import functools

import jax
import jax.numpy as jnp
from jax import lax
from jax.experimental import pallas as pl
from jax.experimental.pallas import tpu as pltpu
from jax.experimental.pallas import tpu_sc as plsc

_TV = 2048
_NEG = -1e30
_DOT_T = (((0,), (1,)), ((), ()))


def _make_sc_gather(n, d):
    info = plsc.get_sparse_core_info()
    nc, ns = info.num_cores, info.num_subcores
    nw = nc * ns
    assert n % nw == 0
    per_w = n // nw
    chunk = 128
    assert per_w % chunk == 0
    nchunk = per_w // chunk
    mesh = plsc.VectorSubcoreMesh(core_axis_name="c", subcore_axis_name="s")

    @functools.partial(
        pl.kernel,
        mesh=mesh,
        out_type=jax.ShapeDtypeStruct((n, d), jnp.float32),
        scratch_types=[
            pltpu.VMEM((per_w,), jnp.int32),
            pltpu.VMEM((per_w, d), jnp.float32),
            pltpu.SemaphoreType.DMA,
        ],
    )
    def gather(table_hbm, idx_hbm, out_hbm, idx_v, rows_v, sem):
        wid = lax.axis_index("s") * nc + lax.axis_index("c")
        base = wid * per_w
        pltpu.sync_copy(idx_hbm.at[pl.ds(base, per_w)], idx_v)
        copies = [
            pltpu.async_copy(
                table_hbm.at[idx_v.at[pl.ds(c * chunk, chunk)]],
                rows_v.at[pl.ds(c * chunk, chunk)],
                sem,
            )
            for c in range(nchunk)
        ]
        for cp in copies:
            cp.wait()
        pltpu.sync_copy(rows_v, out_hbm.at[pl.ds(base, per_w)])

    return gather


def _h_body(x_ref, w1_ref, b1_ref, h2_ref, hm_ref, hacc_ref, *, nsteps, cpg, k2):
    c = pl.program_id(0)

    @pl.when(c == 0)
    def _():
        hacc_ref[...] = jnp.zeros(hacc_ref.shape, jnp.float32)

    b = hacc_ref.shape[0]
    acc = hacc_ref[...]
    for i in range(cpg):
        acc += jnp.dot(x_ref[i * b:(i + 1) * b, :], w1_ref[i],
                       preferred_element_type=jnp.float32)
    hacc_ref[...] = acc

    @pl.when(c == nsteps - 1)
    def _():
        h = jnp.maximum(acc + b1_ref[...], 0.0)
        extra = lax.broadcasted_iota(jnp.int32, (b, k2 - h.shape[1]), 1)
        h2 = jnp.concatenate([h, jnp.where(extra == 0, 1.0, 0.0)], axis=1)
        h2_ref[...] = h2
        hsq = jnp.sum(h2 * h2, axis=1, keepdims=True)
        hm = jnp.sqrt(jnp.max(hsq, axis=0, keepdims=True))
        hm_ref[...] = jnp.broadcast_to(hm, hm_ref.shape)


def _pass1_body(h2_ref, w2_ref, mc_ref, hm_ref, lse_ref, u_ref, s_ref, *, nv):
    j = pl.program_id(0)
    lt = lax.dot_general(w2_ref[...], h2_ref[...], _DOT_T,
                         preferred_element_type=jnp.float32)
    ub = mc_ref[0, 0:1, 0:1] * hm_ref[0:1, 0:1]
    s_j = jnp.sum(jnp.exp(lt - ub), axis=0, keepdims=True)

    @pl.when(j == 0)
    def _():
        u_ref[...] = jnp.full(u_ref.shape, _NEG, jnp.float32)
        s_ref[...] = jnp.zeros(s_ref.shape, jnp.float32)

    u_old = u_ref[0:1, 0:1]
    u_new = jnp.maximum(u_old, ub)
    s_new = (s_ref[...] * jnp.exp(u_old - u_new)
             + s_j * jnp.exp(ub - u_new))
    u_ref[...] = jnp.broadcast_to(u_new, u_ref.shape)
    s_ref[...] = s_new

    @pl.when(j == nv - 1)
    def _():
        lse = u_new + jnp.log(s_new)
        lse_ref[...] = jnp.broadcast_to(lse, lse_ref.shape)


def _pass2_body(h2_ref, w2_ref, lse_ref, out_ref):
    lt = lax.dot_general(w2_ref[...], h2_ref[...], _DOT_T,
                         preferred_element_type=jnp.float32)
    out_ref[...] = lt - lse_ref[:1, :]


def _tc_fused(rows, W1p3, b1, W2a, mc, vocab):
    ctx = W1p3.shape[0]
    b = rows.shape[0] // ctx
    k2 = W2a.shape[0]
    hid = W1p3.shape[2]
    dp = W1p3.shape[1]
    tv = _TV
    nv = W2a.shape[1] // tv
    cpg = 4
    nsteps = ctx // cpg

    h2, _hm = pl.pallas_call(
        functools.partial(_h_body, nsteps=nsteps, cpg=cpg, k2=k2),
        grid=(nsteps,),
        in_specs=[
            pl.BlockSpec((cpg * b, dp), lambda c: (c, 0)),
            pl.BlockSpec((cpg, dp, hid), lambda c: (c, 0, 0)),
            pl.BlockSpec((1, hid), lambda c: (0, 0)),
        ],
        out_specs=[
            pl.BlockSpec((b, k2), lambda c: (0, 0)),
            pl.BlockSpec((1, 128), lambda c: (0, 0)),
        ],
        out_shape=[
            jax.ShapeDtypeStruct((b, k2), jnp.float32),
            jax.ShapeDtypeStruct((1, 128), jnp.float32),
        ],
        scratch_shapes=[pltpu.VMEM((b, hid), jnp.float32)],
        compiler_params=pltpu.CompilerParams(
            dimension_semantics=("arbitrary",),
        ),
    )(rows, W1p3, b1.reshape(1, -1))

    lse = pl.pallas_call(
        functools.partial(_pass1_body, nv=nv),
        grid=(nv,),
        in_specs=[
            pl.BlockSpec((b, k2), lambda j: (0, 0)),
            pl.BlockSpec((k2, tv), lambda j: (0, j)),
            pl.BlockSpec((1, 1, 128), lambda j: (j, 0, 0)),
            pl.BlockSpec((1, 128), lambda j: (0, 0)),
        ],
        out_specs=pl.BlockSpec((8, b), lambda j: (0, 0)),
        out_shape=jax.ShapeDtypeStruct((8, b), jnp.float32),
        scratch_shapes=[
            pltpu.VMEM((1, 128), jnp.float32),
            pltpu.VMEM((1, b), jnp.float32),
        ],
        compiler_params=pltpu.CompilerParams(
            dimension_semantics=("arbitrary",),
        ),
    )(h2, W2a, mc, _hm)

    out_t = pl.pallas_call(
        _pass2_body,
        grid=(nv,),
        in_specs=[
            pl.BlockSpec((b, k2), lambda j: (0, 0)),
            pl.BlockSpec((k2, tv), lambda j: (0, j)),
            pl.BlockSpec((8, b), lambda j: (0, 0)),
        ],
        out_specs=pl.BlockSpec((tv, b), lambda j: (j, 0)),
        out_shape=jax.ShapeDtypeStruct((vocab, b), jnp.float32),
        compiler_params=pltpu.CompilerParams(
            dimension_semantics=("arbitrary",),
        ),
    )(h2, W2a, lse)
    return out_t.T


def kernel(seq, emb, W1, b1, W2, b2):
    b, ctx = seq.shape
    d = emb.shape[1]
    hid = W1.shape[1]
    vocab = W2.shape[1]
    tv = _TV
    nv = pl.cdiv(vocab, tv)
    vpad = nv * tv
    k2 = hid + 8

    dp = 128
    emb_p = jnp.pad(emb, ((0, 0), (0, dp - d)))
    W1p3 = jnp.pad(W1.reshape(ctx, d, hid), ((0, 0), (0, dp - d), (0, 0)))

    bias_row = jnp.concatenate(
        [b2[None, :], jnp.full((1, vpad - vocab), _NEG, jnp.float32)], axis=1)
    W2a = jnp.concatenate(
        [jnp.pad(W2, ((0, 0), (0, vpad - vocab))),
         bias_row,
         jnp.zeros((k2 - hid - 1, vpad), jnp.float32)], axis=0)

    colsq = jnp.sum(W2a * W2a, axis=0)
    colsq = jnp.where(jnp.arange(vpad) < vocab, colsq, 0.0)
    mc = jnp.sqrt(jnp.max(colsq.reshape(nv, tv), axis=1))
    mc = jnp.broadcast_to(mc[:, None, None], (nv, 1, 128))

    seq_cm = seq.T.reshape(-1)
    gather = _make_sc_gather(b * ctx, dp)
    rows = gather(emb_p, seq_cm)
    return _tc_fused(rows, W1p3, b1, W2a, mc, vocab)


# speedup vs baseline: 1.0988x; 1.0010x over previous
import functools

import jax
import jax.numpy as jnp
from jax import lax
from jax.experimental import pallas as pl
from jax.experimental.pallas import tpu as pltpu
from jax.experimental.pallas import tpu_sc as plsc

_TV = 2048
_NEG = -1e30
_DOT_T = (((0,), (1,)), ((), ()))


def _make_sc_gather(n, d):
    info = plsc.get_sparse_core_info()
    nc, ns = info.num_cores, info.num_subcores
    nw = nc * ns
    assert n % nw == 0
    per_w = n // nw
    chunk = 128
    assert per_w % chunk == 0
    nchunk = per_w // chunk
    mesh = plsc.VectorSubcoreMesh(core_axis_name="c", subcore_axis_name="s")

    @functools.partial(
        pl.kernel,
        mesh=mesh,
        out_type=jax.ShapeDtypeStruct((n, d), jnp.float32),
        scratch_types=[
            pltpu.VMEM((per_w,), jnp.int32),
            pltpu.VMEM((per_w, d), jnp.float32),
            pltpu.SemaphoreType.DMA,
        ],
    )
    def gather(table_hbm, idx_hbm, out_hbm, idx_v, rows_v, sem):
        wid = lax.axis_index("s") * nc + lax.axis_index("c")
        base = wid * per_w
        pltpu.sync_copy(idx_hbm.at[pl.ds(base, per_w)], idx_v)
        copies = [
            pltpu.async_copy(
                table_hbm.at[idx_v.at[pl.ds(c * chunk, chunk)]],
                rows_v.at[pl.ds(c * chunk, chunk)],
                sem,
            )
            for c in range(nchunk)
        ]
        for cp in copies:
            cp.wait()
        pltpu.sync_copy(rows_v, out_hbm.at[pl.ds(base, per_w)])

    return gather


def _h_body(x_ref, w1_ref, b1_ref, h2_ref, hm_ref, hacc_ref, *, nsteps, cpg, k2):
    c = pl.program_id(0)

    @pl.when(c == 0)
    def _():
        hacc_ref[...] = jnp.zeros(hacc_ref.shape, jnp.float32)

    b = hacc_ref.shape[0]
    acc = hacc_ref[...]
    for i in range(cpg):
        acc += jnp.dot(x_ref[i * b:(i + 1) * b, :], w1_ref[i],
                       preferred_element_type=jnp.float32)
    hacc_ref[...] = acc

    @pl.when(c == nsteps - 1)
    def _():
        h = jnp.maximum(acc + b1_ref[...], 0.0)
        extra = lax.broadcasted_iota(jnp.int32, (b, k2 - h.shape[1]), 1)
        h2 = jnp.concatenate([h, jnp.where(extra == 0, 1.0, 0.0)], axis=1)
        h2_ref[...] = h2
        hsq = jnp.sum(h2 * h2, axis=1, keepdims=True)
        hm = jnp.sqrt(jnp.max(hsq, axis=0, keepdims=True))
        hm_ref[...] = jnp.broadcast_to(hm, hm_ref.shape)


def _pass1_body(h2_ref, w2_ref, mc_ref, hm_ref, lse_ref, u_ref, s_ref, *, nv):
    j = pl.program_id(0)
    lt = lax.dot_general(w2_ref[...], h2_ref[...], _DOT_T,
                         preferred_element_type=jnp.float32)
    ub = mc_ref[0, 0:1, 0:1] * hm_ref[0:1, 0:1]
    s_j = jnp.sum(jnp.exp(lt - ub), axis=0, keepdims=True)

    @pl.when(j == 0)
    def _():
        u_ref[...] = jnp.full(u_ref.shape, _NEG, jnp.float32)
        s_ref[...] = jnp.zeros(s_ref.shape, jnp.float32)

    u_old = u_ref[0:1, 0:1]
    u_new = jnp.maximum(u_old, ub)
    s_new = (s_ref[...] * jnp.exp(u_old - u_new)
             + s_j * jnp.exp(ub - u_new))
    u_ref[...] = jnp.broadcast_to(u_new, u_ref.shape)
    s_ref[...] = s_new

    @pl.when(j == nv - 1)
    def _():
        lse = u_new + jnp.log(s_new)
        lse_ref[...] = jnp.broadcast_to(lse, lse_ref.shape)


def _pass2_body(h2_ref, w2_ref, lse_ref, out_ref):
    lt = lax.dot_general(w2_ref[...], h2_ref[...], _DOT_T,
                         preferred_element_type=jnp.float32)
    out_ref[...] = lt - lse_ref[:1, :]


def _tc_fused(rows, W1p3, b1, W2a, mc, vocab):
    ctx = W1p3.shape[0]
    b = rows.shape[0] // ctx
    k2 = W2a.shape[0]
    hid = W1p3.shape[2]
    dp = W1p3.shape[1]
    tv = _TV
    nv = W2a.shape[1] // tv
    cpg = 4
    nsteps = ctx // cpg

    h2, _hm = pl.pallas_call(
        functools.partial(_h_body, nsteps=nsteps, cpg=cpg, k2=k2),
        grid=(nsteps,),
        in_specs=[
            pl.BlockSpec((cpg * b, dp), lambda c: (c, 0)),
            pl.BlockSpec((cpg, dp, hid), lambda c: (c, 0, 0)),
            pl.BlockSpec((1, hid), lambda c: (0, 0)),
        ],
        out_specs=[
            pl.BlockSpec((b, k2), lambda c: (0, 0)),
            pl.BlockSpec((1, 128), lambda c: (0, 0)),
        ],
        out_shape=[
            jax.ShapeDtypeStruct((b, k2), jnp.float32),
            jax.ShapeDtypeStruct((1, 128), jnp.float32),
        ],
        scratch_shapes=[pltpu.VMEM((b, hid), jnp.float32)],
        compiler_params=pltpu.CompilerParams(
            dimension_semantics=("arbitrary",),
        ),
    )(rows, W1p3, b1.reshape(1, -1))

    lse = pl.pallas_call(
        functools.partial(_pass1_body, nv=nv),
        grid=(nv,),
        in_specs=[
            pl.BlockSpec((b, k2), lambda j: (0, 0)),
            pl.BlockSpec((k2, tv), lambda j: (0, j)),
            pl.BlockSpec((1, 1, 128), lambda j: (j, 0, 0)),
            pl.BlockSpec((1, 128), lambda j: (0, 0)),
        ],
        out_specs=pl.BlockSpec((8, b), lambda j: (0, 0)),
        out_shape=jax.ShapeDtypeStruct((8, b), jnp.float32),
        scratch_shapes=[
            pltpu.VMEM((1, 128), jnp.float32),
            pltpu.VMEM((1, b), jnp.float32),
        ],
        compiler_params=pltpu.CompilerParams(
            dimension_semantics=("arbitrary",),
        ),
    )(h2, W2a, mc, _hm)

    out_t = pl.pallas_call(
        _pass2_body,
        grid=(nv,),
        in_specs=[
            pl.BlockSpec((b, k2), lambda j: (0, 0)),
            pl.BlockSpec((k2, tv), lambda j: (0, j)),
            pl.BlockSpec((8, b), lambda j: (0, 0)),
        ],
        out_specs=pl.BlockSpec((tv, b), lambda j: (j, 0)),
        out_shape=jax.ShapeDtypeStruct((vocab, b), jnp.float32),
        compiler_params=pltpu.CompilerParams(
            dimension_semantics=("arbitrary",),
        ),
    )(h2, W2a, lse)
    return out_t.T


def kernel(seq, emb, W1, b1, W2, b2):
    b, ctx = seq.shape
    d = emb.shape[1]
    hid = W1.shape[1]
    vocab = W2.shape[1]
    tv = _TV
    nv = pl.cdiv(vocab, tv)
    vpad = nv * tv
    k2 = hid + 8

    dp = 128
    emb_p = jnp.pad(emb, ((0, 0), (0, dp - d)))

    seq_cm = seq.T.reshape(-1)
    gather = _make_sc_gather(b * ctx, dp)
    rows = gather(emb_p, seq_cm)

    W1p3 = jnp.pad(W1.reshape(ctx, d, hid), ((0, 0), (0, dp - d), (0, 0)))

    bias_row = jnp.concatenate(
        [b2[None, :], jnp.full((1, vpad - vocab), _NEG, jnp.float32)], axis=1)
    W2a = jnp.concatenate(
        [jnp.pad(W2, ((0, 0), (0, vpad - vocab))),
         bias_row,
         jnp.zeros((k2 - hid - 1, vpad), jnp.float32)], axis=0)

    colsq = jnp.sum(W2a * W2a, axis=0)
    colsq = jnp.where(jnp.arange(vpad) < vocab, colsq, 0.0)
    mc = jnp.sqrt(jnp.max(colsq.reshape(nv, tv), axis=1))
    mc = jnp.broadcast_to(mc[:, None, None], (nv, 1, 128))

    return _tc_fused(rows, W1p3, b1, W2a, mc, vocab)


# speedup vs baseline: 1.1186x; 1.0181x over previous
import functools

import jax
import jax.numpy as jnp
from jax import lax
from jax.experimental import pallas as pl
from jax.experimental.pallas import tpu as pltpu
from jax.experimental.pallas import tpu_sc as plsc

_TV = 3072
_NEG = -1e30
_DOT_T = (((0,), (1,)), ((), ()))


def _make_sc_gather(n, d):
    info = plsc.get_sparse_core_info()
    nc, ns = info.num_cores, info.num_subcores
    nw = nc * ns
    assert n % nw == 0
    per_w = n // nw
    chunk = 128
    assert per_w % chunk == 0
    nchunk = per_w // chunk
    mesh = plsc.VectorSubcoreMesh(core_axis_name="c", subcore_axis_name="s")

    @functools.partial(
        pl.kernel,
        mesh=mesh,
        out_type=jax.ShapeDtypeStruct((n, d), jnp.float32),
        scratch_types=[
            pltpu.VMEM((per_w,), jnp.int32),
            pltpu.VMEM((per_w, d), jnp.float32),
            pltpu.SemaphoreType.DMA,
        ],
    )
    def gather(table_hbm, idx_hbm, out_hbm, idx_v, rows_v, sem):
        wid = lax.axis_index("s") * nc + lax.axis_index("c")
        base = wid * per_w
        pltpu.sync_copy(idx_hbm.at[pl.ds(base, per_w)], idx_v)
        copies = [
            pltpu.async_copy(
                table_hbm.at[idx_v.at[pl.ds(c * chunk, chunk)]],
                rows_v.at[pl.ds(c * chunk, chunk)],
                sem,
            )
            for c in range(nchunk)
        ]
        for cp in copies:
            cp.wait()
        pltpu.sync_copy(rows_v, out_hbm.at[pl.ds(base, per_w)])

    return gather


def _h_body(x_ref, w1_ref, b1_ref, h2_ref, hm_ref, hacc_ref, *, nsteps, cpg, k2):
    c = pl.program_id(0)

    @pl.when(c == 0)
    def _():
        hacc_ref[...] = jnp.zeros(hacc_ref.shape, jnp.float32)

    b = hacc_ref.shape[0]
    acc = hacc_ref[...]
    for i in range(cpg):
        acc += jnp.dot(x_ref[i * b:(i + 1) * b, :], w1_ref[i],
                       preferred_element_type=jnp.float32)
    hacc_ref[...] = acc

    @pl.when(c == nsteps - 1)
    def _():
        h = jnp.maximum(acc + b1_ref[...], 0.0)
        extra = lax.broadcasted_iota(jnp.int32, (b, k2 - h.shape[1]), 1)
        h2 = jnp.concatenate([h, jnp.where(extra == 0, 1.0, 0.0)], axis=1)
        h2_ref[...] = h2
        hsq = jnp.sum(h2 * h2, axis=1, keepdims=True)
        hm = jnp.sqrt(jnp.max(hsq, axis=0, keepdims=True))
        hm_ref[...] = jnp.broadcast_to(hm, hm_ref.shape)


def _pass1_body(h2_ref, w2_ref, mc_ref, hm_ref, lse_ref, u_ref, s_ref, *, nv):
    j = pl.program_id(0)
    lt = lax.dot_general(w2_ref[...], h2_ref[...], _DOT_T,
                         preferred_element_type=jnp.float32)
    ub = mc_ref[0, 0:1, 0:1] * hm_ref[0:1, 0:1]
    s_j = jnp.sum(jnp.exp(lt - ub), axis=0, keepdims=True)

    @pl.when(j == 0)
    def _():
        u_ref[...] = jnp.full(u_ref.shape, _NEG, jnp.float32)
        s_ref[...] = jnp.zeros(s_ref.shape, jnp.float32)

    u_old = u_ref[0:1, 0:1]
    u_new = jnp.maximum(u_old, ub)
    s_new = (s_ref[...] * jnp.exp(u_old - u_new)
             + s_j * jnp.exp(ub - u_new))
    u_ref[...] = jnp.broadcast_to(u_new, u_ref.shape)
    s_ref[...] = s_new

    @pl.when(j == nv - 1)
    def _():
        lse = u_new + jnp.log(s_new)
        lse_ref[...] = jnp.broadcast_to(lse, lse_ref.shape)


def _pass2_body(h2_ref, w2_ref, lse_ref, out_ref):
    lt = lax.dot_general(w2_ref[...], h2_ref[...], _DOT_T,
                         preferred_element_type=jnp.float32)
    out_ref[...] = lt - lse_ref[:1, :]


def _tc_fused(rows, W1p3, b1, W2a, mc, vocab):
    ctx = W1p3.shape[0]
    b = rows.shape[0] // ctx
    k2 = W2a.shape[0]
    hid = W1p3.shape[2]
    dp = W1p3.shape[1]
    tv = _TV
    nv = W2a.shape[1] // tv
    cpg = 4
    nsteps = ctx // cpg

    h2, _hm = pl.pallas_call(
        functools.partial(_h_body, nsteps=nsteps, cpg=cpg, k2=k2),
        grid=(nsteps,),
        in_specs=[
            pl.BlockSpec((cpg * b, dp), lambda c: (c, 0)),
            pl.BlockSpec((cpg, dp, hid), lambda c: (c, 0, 0)),
            pl.BlockSpec((1, hid), lambda c: (0, 0)),
        ],
        out_specs=[
            pl.BlockSpec((b, k2), lambda c: (0, 0)),
            pl.BlockSpec((1, 128), lambda c: (0, 0)),
        ],
        out_shape=[
            jax.ShapeDtypeStruct((b, k2), jnp.float32),
            jax.ShapeDtypeStruct((1, 128), jnp.float32),
        ],
        scratch_shapes=[pltpu.VMEM((b, hid), jnp.float32)],
        compiler_params=pltpu.CompilerParams(
            dimension_semantics=("arbitrary",),
        ),
    )(rows, W1p3, b1.reshape(1, -1))

    lse = pl.pallas_call(
        functools.partial(_pass1_body, nv=nv),
        grid=(nv,),
        in_specs=[
            pl.BlockSpec((b, k2), lambda j: (0, 0)),
            pl.BlockSpec((k2, tv), lambda j: (0, j)),
            pl.BlockSpec((1, 1, 128), lambda j: (j, 0, 0)),
            pl.BlockSpec((1, 128), lambda j: (0, 0)),
        ],
        out_specs=pl.BlockSpec((8, b), lambda j: (0, 0)),
        out_shape=jax.ShapeDtypeStruct((8, b), jnp.float32),
        scratch_shapes=[
            pltpu.VMEM((1, 128), jnp.float32),
            pltpu.VMEM((1, b), jnp.float32),
        ],
        compiler_params=pltpu.CompilerParams(
            dimension_semantics=("arbitrary",),
        ),
    )(h2, W2a, mc, _hm)

    out_t = pl.pallas_call(
        _pass2_body,
        grid=(nv,),
        in_specs=[
            pl.BlockSpec((b, k2), lambda j: (0, 0)),
            pl.BlockSpec((k2, tv), lambda j: (0, j)),
            pl.BlockSpec((8, b), lambda j: (0, 0)),
        ],
        out_specs=pl.BlockSpec((tv, b), lambda j: (j, 0)),
        out_shape=jax.ShapeDtypeStruct((vocab, b), jnp.float32),
        compiler_params=pltpu.CompilerParams(
            dimension_semantics=("arbitrary",),
        ),
    )(h2, W2a, lse)
    return out_t.T


def kernel(seq, emb, W1, b1, W2, b2):
    b, ctx = seq.shape
    d = emb.shape[1]
    hid = W1.shape[1]
    vocab = W2.shape[1]
    tv = _TV
    nv = pl.cdiv(vocab, tv)
    vpad = nv * tv
    k2 = hid + 8

    dp = 128
    emb_p = jnp.pad(emb, ((0, 0), (0, dp - d)))

    seq_cm = seq.T.reshape(-1)
    gather = _make_sc_gather(b * ctx, dp)
    rows = gather(emb_p, seq_cm)

    W1p3 = jnp.pad(W1.reshape(ctx, d, hid), ((0, 0), (0, dp - d), (0, 0)))

    bias_row = jnp.concatenate(
        [b2[None, :], jnp.full((1, vpad - vocab), _NEG, jnp.float32)], axis=1)
    W2a = jnp.concatenate(
        [jnp.pad(W2, ((0, 0), (0, vpad - vocab))),
         bias_row,
         jnp.zeros((k2 - hid - 1, vpad), jnp.float32)], axis=0)

    colsq = jnp.sum(W2a * W2a, axis=0)
    colsq = jnp.where(jnp.arange(vpad) < vocab, colsq, 0.0)
    mc = jnp.sqrt(jnp.max(colsq.reshape(nv, tv), axis=1))
    mc = jnp.broadcast_to(mc[:, None, None], (nv, 1, 128))

    return _tc_fused(rows, W1p3, b1, W2a, mc, vocab)


# speedup vs baseline: 1.1263x; 1.0068x over previous
import functools

import jax
import jax.numpy as jnp
from jax import lax
from jax.experimental import pallas as pl
from jax.experimental.pallas import tpu as pltpu
from jax.experimental.pallas import tpu_sc as plsc

_TV = 4096
_NEG = -1e30
_DOT_T = (((0,), (1,)), ((), ()))


def _make_sc_gather(n, d):
    info = plsc.get_sparse_core_info()
    nc, ns = info.num_cores, info.num_subcores
    nw = nc * ns
    assert n % nw == 0
    per_w = n // nw
    chunk = 128
    assert per_w % chunk == 0
    nchunk = per_w // chunk
    mesh = plsc.VectorSubcoreMesh(core_axis_name="c", subcore_axis_name="s")

    @functools.partial(
        pl.kernel,
        mesh=mesh,
        out_type=jax.ShapeDtypeStruct((n, d), jnp.float32),
        scratch_types=[
            pltpu.VMEM((per_w,), jnp.int32),
            pltpu.VMEM((per_w, d), jnp.float32),
            pltpu.SemaphoreType.DMA,
        ],
    )
    def gather(table_hbm, idx_hbm, out_hbm, idx_v, rows_v, sem):
        wid = lax.axis_index("s") * nc + lax.axis_index("c")
        base = wid * per_w
        pltpu.sync_copy(idx_hbm.at[pl.ds(base, per_w)], idx_v)
        copies = [
            pltpu.async_copy(
                table_hbm.at[idx_v.at[pl.ds(c * chunk, chunk)]],
                rows_v.at[pl.ds(c * chunk, chunk)],
                sem,
            )
            for c in range(nchunk)
        ]
        for cp in copies:
            cp.wait()
        pltpu.sync_copy(rows_v, out_hbm.at[pl.ds(base, per_w)])

    return gather


def _h_body(x_ref, w1_ref, b1_ref, h2_ref, hm_ref, hacc_ref, *, nsteps, cpg, k2):
    c = pl.program_id(0)

    @pl.when(c == 0)
    def _():
        hacc_ref[...] = jnp.zeros(hacc_ref.shape, jnp.float32)

    b = hacc_ref.shape[0]
    acc = hacc_ref[...]
    for i in range(cpg):
        acc += jnp.dot(x_ref[i * b:(i + 1) * b, :], w1_ref[i],
                       preferred_element_type=jnp.float32)
    hacc_ref[...] = acc

    @pl.when(c == nsteps - 1)
    def _():
        h = jnp.maximum(acc + b1_ref[...], 0.0)
        extra = lax.broadcasted_iota(jnp.int32, (b, k2 - h.shape[1]), 1)
        h2 = jnp.concatenate([h, jnp.where(extra == 0, 1.0, 0.0)], axis=1)
        h2_ref[...] = h2
        hsq = jnp.sum(h2 * h2, axis=1, keepdims=True)
        hm = jnp.sqrt(jnp.max(hsq, axis=0, keepdims=True))
        hm_ref[...] = jnp.broadcast_to(hm, hm_ref.shape)


def _pass1_body(h2_ref, w2_ref, mc_ref, hm_ref, lse_ref, u_ref, s_ref, *, nv):
    j = pl.program_id(0)
    lt = lax.dot_general(w2_ref[...], h2_ref[...], _DOT_T,
                         preferred_element_type=jnp.float32)
    ub = mc_ref[0, 0:1, 0:1] * hm_ref[0:1, 0:1]
    s_j = jnp.sum(jnp.exp(lt - ub), axis=0, keepdims=True)

    @pl.when(j == 0)
    def _():
        u_ref[...] = jnp.full(u_ref.shape, _NEG, jnp.float32)
        s_ref[...] = jnp.zeros(s_ref.shape, jnp.float32)

    u_old = u_ref[0:1, 0:1]
    u_new = jnp.maximum(u_old, ub)
    s_new = (s_ref[...] * jnp.exp(u_old - u_new)
             + s_j * jnp.exp(ub - u_new))
    u_ref[...] = jnp.broadcast_to(u_new, u_ref.shape)
    s_ref[...] = s_new

    @pl.when(j == nv - 1)
    def _():
        lse = u_new + jnp.log(s_new)
        lse_ref[...] = jnp.broadcast_to(lse, lse_ref.shape)


def _pass2_body(h2_ref, w2_ref, lse_ref, out_ref):
    lt = lax.dot_general(w2_ref[...], h2_ref[...], _DOT_T,
                         preferred_element_type=jnp.float32)
    out_ref[...] = lt - lse_ref[:1, :]


def _tc_fused(rows, W1p3, b1, W2a, mc, vocab):
    ctx = W1p3.shape[0]
    b = rows.shape[0] // ctx
    k2 = W2a.shape[0]
    hid = W1p3.shape[2]
    dp = W1p3.shape[1]
    tv = _TV
    nv = W2a.shape[1] // tv
    cpg = 4
    nsteps = ctx // cpg

    h2, _hm = pl.pallas_call(
        functools.partial(_h_body, nsteps=nsteps, cpg=cpg, k2=k2),
        grid=(nsteps,),
        in_specs=[
            pl.BlockSpec((cpg * b, dp), lambda c: (c, 0)),
            pl.BlockSpec((cpg, dp, hid), lambda c: (c, 0, 0)),
            pl.BlockSpec((1, hid), lambda c: (0, 0)),
        ],
        out_specs=[
            pl.BlockSpec((b, k2), lambda c: (0, 0)),
            pl.BlockSpec((1, 128), lambda c: (0, 0)),
        ],
        out_shape=[
            jax.ShapeDtypeStruct((b, k2), jnp.float32),
            jax.ShapeDtypeStruct((1, 128), jnp.float32),
        ],
        scratch_shapes=[pltpu.VMEM((b, hid), jnp.float32)],
        compiler_params=pltpu.CompilerParams(
            dimension_semantics=("arbitrary",),
        ),
    )(rows, W1p3, b1.reshape(1, -1))

    lse = pl.pallas_call(
        functools.partial(_pass1_body, nv=nv),
        grid=(nv,),
        in_specs=[
            pl.BlockSpec((b, k2), lambda j: (0, 0)),
            pl.BlockSpec((k2, tv), lambda j: (0, j)),
            pl.BlockSpec((1, 1, 128), lambda j: (j, 0, 0)),
            pl.BlockSpec((1, 128), lambda j: (0, 0)),
        ],
        out_specs=pl.BlockSpec((8, b), lambda j: (0, 0)),
        out_shape=jax.ShapeDtypeStruct((8, b), jnp.float32),
        scratch_shapes=[
            pltpu.VMEM((1, 128), jnp.float32),
            pltpu.VMEM((1, b), jnp.float32),
        ],
        compiler_params=pltpu.CompilerParams(
            dimension_semantics=("arbitrary",),
        ),
    )(h2, W2a, mc, _hm)

    out_t = pl.pallas_call(
        _pass2_body,
        grid=(nv,),
        in_specs=[
            pl.BlockSpec((b, k2), lambda j: (0, 0)),
            pl.BlockSpec((k2, tv), lambda j: (0, j)),
            pl.BlockSpec((8, b), lambda j: (0, 0)),
        ],
        out_specs=pl.BlockSpec((tv, b), lambda j: (j, 0)),
        out_shape=jax.ShapeDtypeStruct((vocab, b), jnp.float32),
        compiler_params=pltpu.CompilerParams(
            dimension_semantics=("arbitrary",),
        ),
    )(h2, W2a, lse)
    return out_t.T


def kernel(seq, emb, W1, b1, W2, b2):
    b, ctx = seq.shape
    d = emb.shape[1]
    hid = W1.shape[1]
    vocab = W2.shape[1]
    tv = _TV
    nv = pl.cdiv(vocab, tv)
    vpad = nv * tv
    k2 = hid + 8

    dp = 128
    emb_p = jnp.pad(emb, ((0, 0), (0, dp - d)))

    seq_cm = seq.T.reshape(-1)
    gather = _make_sc_gather(b * ctx, dp)
    rows = gather(emb_p, seq_cm)

    W1p3 = jnp.pad(W1.reshape(ctx, d, hid), ((0, 0), (0, dp - d), (0, 0)))

    bias_row = jnp.concatenate(
        [b2[None, :], jnp.full((1, vpad - vocab), _NEG, jnp.float32)], axis=1)
    W2a = jnp.concatenate(
        [jnp.pad(W2, ((0, 0), (0, vpad - vocab))),
         bias_row,
         jnp.zeros((k2 - hid - 1, vpad), jnp.float32)], axis=0)

    colsq = jnp.sum(W2a * W2a, axis=0)
    colsq = jnp.where(jnp.arange(vpad) < vocab, colsq, 0.0)
    mc = jnp.sqrt(jnp.max(colsq.reshape(nv, tv), axis=1))
    mc = jnp.broadcast_to(mc[:, None, None], (nv, 1, 128))

    return _tc_fused(rows, W1p3, b1, W2a, mc, vocab)
